# Initial kernel scaffold; baseline (speedup 1.0000x reference)
import jax
import jax.numpy as jnp
from jax.experimental import pallas as pl


def kernel(h_real, h_imag, iw1, ib1, iw2, ib2, nw1, nb1, nw2, nb2):
    raise NotImplementedError("write your pallas kernel here")


# speedup vs baseline: 1.2060x; 1.2060x over previous
import functools

import jax
import jax.numpy as jnp
from jax import lax
from jax.experimental import pallas as pl

B = 8
K = 32768
H = 128
M = 1024
CHUNK = 2048
NCH = K // CHUNK


def _mlp_in_kernel(hr_ref, hi_ref, mask_ref, w_ref, b_ref, out_ref):
    g = pl.program_id(0)
    x = jnp.where(g < NCH, hr_ref[...], hi_ref[...])
    if mask_ref is not None:
        x = x * mask_ref[...]
    acc = jnp.dot(x, w_ref[...], preferred_element_type=jnp.float32)

    @pl.when(g == 0)
    def _():
        out_ref[...] = b_ref[...] + acc

    @pl.when(g > 0)
    def _():
        out_ref[...] += acc


def _mlp_in(hr, hi, mask, w, b):
    use_mask = mask is not None
    kern = functools.partial(_mlp_in_kernel) if use_mask else (
        lambda hr_ref, hi_ref, w_ref, b_ref, out_ref: _mlp_in_kernel(
            hr_ref, hi_ref, None, w_ref, b_ref, out_ref))
    in_specs = [
        pl.BlockSpec((B, CHUNK), lambda g: (0, jnp.where(g < NCH, g, 0))),
        pl.BlockSpec((B, CHUNK), lambda g: (0, jnp.where(g < NCH, 0, g - NCH))),
    ]
    args = [hr, hi]
    if use_mask:
        in_specs.append(pl.BlockSpec((B, CHUNK), lambda g: (0, lax.rem(g, NCH))))
        args.append(mask)
    in_specs += [
        pl.BlockSpec((CHUNK, H), lambda g: (g, 0)),
        pl.BlockSpec((1, H), lambda g: (0, 0)),
    ]
    args += [w, b.reshape(1, H)]
    out = pl.pallas_call(
        kern,
        grid=(2 * NCH,),
        in_specs=in_specs,
        out_specs=pl.BlockSpec((B, H), lambda g: (0, 0)),
        out_shape=jax.ShapeDtypeStruct((B, H), jnp.float32),
    )(*args)
    return jnp.maximum(out, 0.0)


def _mlp_out_kernel(h_ref, w_ref, b_ref, out_ref):
    out_ref[...] = b_ref[...] + jnp.dot(
        h_ref[...], w_ref[...], preferred_element_type=jnp.float32)


def _mlp_out(h, w, b):
    return pl.pallas_call(
        _mlp_out_kernel,
        grid=(NCH,),
        in_specs=[
            pl.BlockSpec((B, H), lambda g: (0, 0)),
            pl.BlockSpec((H, CHUNK), lambda g: (0, g)),
            pl.BlockSpec((1, CHUNK), lambda g: (0, g)),
        ],
        out_specs=pl.BlockSpec((B, CHUNK), lambda g: (0, g)),
        out_shape=jax.ShapeDtypeStruct((B, K), jnp.float32),
    )(h, w, b.reshape(1, K))


def _ce_pass(v, idx, s, dirsize):
    _, w = v.shape
    pos = lax.broadcasted_iota(jnp.int32, (1, w), 1)
    low = (pos & s) == 0
    desc = (pos & dirsize) == 0
    pv = jnp.where(low, jnp.roll(v, -s, axis=1), jnp.roll(v, s, axis=1))
    pi = jnp.where(low, jnp.roll(idx, -s, axis=1), jnp.roll(idx, s, axis=1))
    before = (v > pv) | ((v == pv) & (idx < pi))
    keep = before == (low == desc)
    return jnp.where(keep, v, pv), jnp.where(keep, idx, pi)


def _winner_pass(v, idx, half):
    _, w = v.shape
    pos = lax.broadcasted_iota(jnp.int32, (1, w), 1)
    low = (pos & half) == 0
    pv = jnp.where(low, jnp.roll(v, -half, axis=1), jnp.roll(v, half, axis=1))
    pi = jnp.where(low, jnp.roll(idx, -half, axis=1), jnp.roll(idx, half, axis=1))
    before = (v > pv) | ((v == pv) & (idx < pi))
    keep = before == low
    return jnp.where(keep, v, pv), jnp.where(keep, idx, pi)


def _drop_high_halves(x, half):
    b, w = x.shape
    nb = w // (2 * half)
    return x.reshape(b * nb, 2, half)[:, 0, :].reshape(b, w // 2)


def _topk_kernel(scores_ref, idx_ref, mask_ref, vals_ref):
    v = scores_ref[...]
    idx = lax.broadcasted_iota(jnp.int32, (B, K), 1)

    for k in range(1, 11):
        for j in reversed(range(k)):
            v, idx = _ce_pass(v, idx, 1 << j, 1 << k)

    w = K
    while w > M:
        v, idx = _winner_pass(v, idx, M)
        v = _drop_high_halves(v, M)
        idx = _drop_high_halves(idx, M)
        w //= 2
        for j in reversed(range(10)):
            v, idx = _ce_pass(v, idx, 1 << j, M)

    idx_ref[...] = idx
    vals_ref[...] = v

    sv = scores_ref[...]
    t = v[:, M - 1:M]
    gt = sv > t
    eq = sv == t
    c_gt = jnp.sum(gt.astype(jnp.int32), axis=1, keepdims=True)
    tie = eq.astype(jnp.int32)
    sh = 1
    while sh < K:
        pos = lax.broadcasted_iota(jnp.int32, (1, K), 1)
        shifted = jnp.roll(tie, sh, axis=1)
        tie = tie + jnp.where(pos >= sh, shifted, 0)
        sh *= 2
    sel = gt | (eq & (tie <= (M - c_gt)))
    mask_ref[...] = sel.astype(jnp.float32)


def _topk(scores):
    return pl.pallas_call(
        _topk_kernel,
        in_specs=[pl.BlockSpec((B, K), lambda: (0, 0))],
        out_specs=[
            pl.BlockSpec((B, M), lambda: (0, 0)),
            pl.BlockSpec((B, K), lambda: (0, 0)),
            pl.BlockSpec((B, M), lambda: (0, 0)),
        ],
        out_shape=[
            jax.ShapeDtypeStruct((B, M), jnp.int32),
            jax.ShapeDtypeStruct((B, K), jnp.float32),
            jax.ShapeDtypeStruct((B, M), jnp.float32),
        ],
    )(scores)


def kernel(h_real, h_imag, iw1, ib1, iw2, ib2, nw1, nb1, nw2, nb2):
    hidden = _mlp_in(h_real, h_imag, None, iw1, ib1)
    scores = _mlp_out(hidden, iw2, ib2)
    sel_idx, mask, _vals = _topk(scores)
    hidden2 = _mlp_in(h_real, h_imag, mask, nw1, nb1)
    output = _mlp_out(hidden2, nw2, nb2)
    return (output, sel_idx)


# speedup vs baseline: 1.2219x; 1.0132x over previous
import functools

import jax
import jax.numpy as jnp
from jax import lax
from jax.experimental import pallas as pl

B = 8
K = 32768
H = 128
M = 1024
CHUNK = 2048
NCH = K // CHUNK


def _mlp_in_kernel(hr_ref, hi_ref, mask_ref, w_ref, b_ref, out_ref):
    g = pl.program_id(0)
    x = jnp.where(g < NCH, hr_ref[...], hi_ref[...])
    if mask_ref is not None:
        x = x * mask_ref[...]
    acc = jnp.dot(x, w_ref[...], preferred_element_type=jnp.float32)

    @pl.when(g == 0)
    def _():
        out_ref[...] = b_ref[...] + acc

    @pl.when(g > 0)
    def _():
        out_ref[...] += acc


def _mlp_in(hr, hi, mask, w, b):
    use_mask = mask is not None
    kern = functools.partial(_mlp_in_kernel) if use_mask else (
        lambda hr_ref, hi_ref, w_ref, b_ref, out_ref: _mlp_in_kernel(
            hr_ref, hi_ref, None, w_ref, b_ref, out_ref))
    in_specs = [
        pl.BlockSpec((B, CHUNK), lambda g: (0, jnp.where(g < NCH, g, 0))),
        pl.BlockSpec((B, CHUNK), lambda g: (0, jnp.where(g < NCH, 0, g - NCH))),
    ]
    args = [hr, hi]
    if use_mask:
        in_specs.append(pl.BlockSpec((B, CHUNK), lambda g: (0, lax.rem(g, NCH))))
        args.append(mask)
    in_specs += [
        pl.BlockSpec((CHUNK, H), lambda g: (g, 0)),
        pl.BlockSpec((1, H), lambda g: (0, 0)),
    ]
    args += [w, b.reshape(1, H)]
    out = pl.pallas_call(
        kern,
        grid=(2 * NCH,),
        in_specs=in_specs,
        out_specs=pl.BlockSpec((B, H), lambda g: (0, 0)),
        out_shape=jax.ShapeDtypeStruct((B, H), jnp.float32),
    )(*args)
    return jnp.maximum(out, 0.0)


def _mlp_out_kernel(h_ref, w_ref, b_ref, out_ref):
    out_ref[...] = b_ref[...] + jnp.dot(
        h_ref[...], w_ref[...], preferred_element_type=jnp.float32)


def _mlp_out(h, w, b):
    return pl.pallas_call(
        _mlp_out_kernel,
        grid=(NCH,),
        in_specs=[
            pl.BlockSpec((B, H), lambda g: (0, 0)),
            pl.BlockSpec((H, CHUNK), lambda g: (0, g)),
            pl.BlockSpec((1, CHUNK), lambda g: (0, g)),
        ],
        out_specs=pl.BlockSpec((B, CHUNK), lambda g: (0, g)),
        out_shape=jax.ShapeDtypeStruct((B, K), jnp.float32),
    )(h, w, b.reshape(1, K))


def _ce_pass(v, idx, s, dirsize):
    _, w = v.shape
    pos = lax.broadcasted_iota(jnp.int32, (1, w), 1)
    low = (pos & s) == 0
    desc = (pos & dirsize) == 0
    pv = jnp.where(low, jnp.roll(v, -s, axis=1), jnp.roll(v, s, axis=1))
    pi = jnp.where(low, jnp.roll(idx, -s, axis=1), jnp.roll(idx, s, axis=1))
    before = (v > pv) | ((v == pv) & (idx < pi))
    keep = before == (low == desc)
    return jnp.where(keep, v, pv), jnp.where(keep, idx, pi)


def _winner_pass(v, idx, half):
    _, w = v.shape
    pos = lax.broadcasted_iota(jnp.int32, (1, w), 1)
    low = (pos & half) == 0
    pv = jnp.where(low, jnp.roll(v, -half, axis=1), jnp.roll(v, half, axis=1))
    pi = jnp.where(low, jnp.roll(idx, -half, axis=1), jnp.roll(idx, half, axis=1))
    before = (v > pv) | ((v == pv) & (idx < pi))
    keep = before == low
    return jnp.where(keep, v, pv), jnp.where(keep, idx, pi)


def _drop_high_halves(x, half):
    b, w = x.shape
    nb = w // (2 * half)
    return x.reshape(b * nb, 2, half)[:, 0, :].reshape(b, w // 2)


def _topk_kernel(scores_ref, idx_ref, mask_ref, vals_ref):
    v = scores_ref[...]
    idx = lax.broadcasted_iota(jnp.int32, (B, K), 1)

    for k in range(1, 11):
        for j in reversed(range(k)):
            v, idx = _ce_pass(v, idx, 1 << j, 1 << k)

    w = K
    while w > M:
        v, idx = _winner_pass(v, idx, M)
        v = _drop_high_halves(v, M)
        idx = _drop_high_halves(idx, M)
        w //= 2
        for j in reversed(range(10)):
            v, idx = _ce_pass(v, idx, 1 << j, M)

    idx_ref[...] = idx
    vals_ref[...] = v

    sv = scores_ref[...]
    t = v[:, M - 1:M]
    ti = idx[:, M - 1:M]
    pos = lax.broadcasted_iota(jnp.int32, (B, K), 1)
    sel = (sv > t) | ((sv == t) & (pos <= ti))
    mask_ref[...] = sel.astype(jnp.float32)


def _topk(scores):
    return pl.pallas_call(
        _topk_kernel,
        in_specs=[pl.BlockSpec((B, K), lambda: (0, 0))],
        out_specs=[
            pl.BlockSpec((B, M), lambda: (0, 0)),
            pl.BlockSpec((B, K), lambda: (0, 0)),
            pl.BlockSpec((B, M), lambda: (0, 0)),
        ],
        out_shape=[
            jax.ShapeDtypeStruct((B, M), jnp.int32),
            jax.ShapeDtypeStruct((B, K), jnp.float32),
            jax.ShapeDtypeStruct((B, M), jnp.float32),
        ],
    )(scores)


def kernel(h_real, h_imag, iw1, ib1, iw2, ib2, nw1, nb1, nw2, nb2):
    hidden = _mlp_in(h_real, h_imag, None, iw1, ib1)
    scores = _mlp_out(hidden, iw2, ib2)
    sel_idx, mask, _vals = _topk(scores)
    hidden2 = _mlp_in(h_real, h_imag, mask, nw1, nb1)
    output = _mlp_out(hidden2, nw2, nb2)
    return (output, sel_idx)


# speedup vs baseline: 1.4042x; 1.1492x over previous
import functools

import jax
import jax.numpy as jnp
from jax import lax
from jax.experimental import pallas as pl

B = 8
K = 32768
H = 128
M = 1024
CHUNK = 4096
NCH = K // CHUNK


def _mlp_in_kernel(hr_ref, hi_ref, mask_ref, w_ref, b_ref, out_ref):
    g = pl.program_id(0)
    x = jnp.where(g < NCH, hr_ref[...], hi_ref[...])
    if mask_ref is not None:
        x = x * mask_ref[...]
    acc = jnp.dot(x, w_ref[...], preferred_element_type=jnp.float32)

    @pl.when(g == 0)
    def _():
        out_ref[...] = b_ref[...] + acc

    @pl.when(g > 0)
    def _():
        out_ref[...] += acc


def _mlp_in(hr, hi, mask, w, b):
    use_mask = mask is not None
    kern = functools.partial(_mlp_in_kernel) if use_mask else (
        lambda hr_ref, hi_ref, w_ref, b_ref, out_ref: _mlp_in_kernel(
            hr_ref, hi_ref, None, w_ref, b_ref, out_ref))
    in_specs = [
        pl.BlockSpec((B, CHUNK), lambda g: (0, jnp.where(g < NCH, g, 0))),
        pl.BlockSpec((B, CHUNK), lambda g: (0, jnp.where(g < NCH, 0, g - NCH))),
    ]
    args = [hr, hi]
    if use_mask:
        in_specs.append(pl.BlockSpec((B, CHUNK), lambda g: (0, lax.rem(g, NCH))))
        args.append(mask)
    in_specs += [
        pl.BlockSpec((CHUNK, H), lambda g: (g, 0)),
        pl.BlockSpec((1, H), lambda g: (0, 0)),
    ]
    args += [w, b.reshape(1, H)]
    out = pl.pallas_call(
        kern,
        grid=(2 * NCH,),
        in_specs=in_specs,
        out_specs=pl.BlockSpec((B, H), lambda g: (0, 0)),
        out_shape=jax.ShapeDtypeStruct((B, H), jnp.float32),
    )(*args)
    return jnp.maximum(out, 0.0)


def _mlp_out_kernel(h_ref, w_ref, b_ref, out_ref):
    out_ref[...] = b_ref[...] + jnp.dot(
        h_ref[...], w_ref[...], preferred_element_type=jnp.float32)


def _mlp_out(h, w, b):
    return pl.pallas_call(
        _mlp_out_kernel,
        grid=(NCH,),
        in_specs=[
            pl.BlockSpec((B, H), lambda g: (0, 0)),
            pl.BlockSpec((H, CHUNK), lambda g: (0, g)),
            pl.BlockSpec((1, CHUNK), lambda g: (0, g)),
        ],
        out_specs=pl.BlockSpec((B, CHUNK), lambda g: (0, g)),
        out_shape=jax.ShapeDtypeStruct((B, K), jnp.float32),
    )(h, w, b.reshape(1, K))


def _ce_pass(v, idx, s, dirsize):
    _, w = v.shape
    pos = lax.broadcasted_iota(jnp.int32, (1, w), 1)
    low = (pos & s) == 0
    desc = (pos & dirsize) == 0
    pv = jnp.where(low, jnp.roll(v, -s, axis=1), jnp.roll(v, s, axis=1))
    pi = jnp.where(low, jnp.roll(idx, -s, axis=1), jnp.roll(idx, s, axis=1))
    before = (v > pv) | ((v == pv) & (idx < pi))
    keep = before == (low == desc)
    return jnp.where(keep, v, pv), jnp.where(keep, idx, pi)


def _winner_pass(v, idx, half):
    _, w = v.shape
    pos = lax.broadcasted_iota(jnp.int32, (1, w), 1)
    low = (pos & half) == 0
    pv = jnp.where(low, jnp.roll(v, -half, axis=1), jnp.roll(v, half, axis=1))
    pi = jnp.where(low, jnp.roll(idx, -half, axis=1), jnp.roll(idx, half, axis=1))
    before = (v > pv) | ((v == pv) & (idx < pi))
    keep = before == low
    return jnp.where(keep, v, pv), jnp.where(keep, idx, pi)


def _drop_high_halves(x, half):
    b, w = x.shape
    nb = w // (2 * half)
    return x.reshape(b * nb, 2, half)[:, 0, :].reshape(b, w // 2)


def _topk_kernel(scores_ref, idx_ref, mask_ref, vals_ref):
    v = scores_ref[...]
    idx = lax.broadcasted_iota(jnp.int32, (B, K), 1)

    for k in range(1, 11):
        for j in reversed(range(k)):
            v, idx = _ce_pass(v, idx, 1 << j, 1 << k)

    w = K
    while w > M:
        v, idx = _winner_pass(v, idx, M)
        v = _drop_high_halves(v, M)
        idx = _drop_high_halves(idx, M)
        w //= 2
        for j in reversed(range(10)):
            v, idx = _ce_pass(v, idx, 1 << j, M)

    idx_ref[...] = idx
    vals_ref[...] = v

    sv = scores_ref[...]
    t = v[:, M - 1:M]
    ti = idx[:, M - 1:M]
    pos = lax.broadcasted_iota(jnp.int32, (B, K), 1)
    sel = (sv > t) | ((sv == t) & (pos <= ti))
    mask_ref[...] = sel.astype(jnp.float32)


def _topk(scores):
    return pl.pallas_call(
        _topk_kernel,
        in_specs=[pl.BlockSpec((B, K), lambda: (0, 0))],
        out_specs=[
            pl.BlockSpec((B, M), lambda: (0, 0)),
            pl.BlockSpec((B, K), lambda: (0, 0)),
            pl.BlockSpec((B, M), lambda: (0, 0)),
        ],
        out_shape=[
            jax.ShapeDtypeStruct((B, M), jnp.int32),
            jax.ShapeDtypeStruct((B, K), jnp.float32),
            jax.ShapeDtypeStruct((B, M), jnp.float32),
        ],
    )(scores)


def kernel(h_real, h_imag, iw1, ib1, iw2, ib2, nw1, nb1, nw2, nb2):
    hidden = _mlp_in(h_real, h_imag, None, iw1, ib1)
    scores = _mlp_out(hidden, iw2, ib2)
    sel_idx, mask, _vals = _topk(scores)
    hidden2 = _mlp_in(h_real, h_imag, mask, nw1, nb1)
    output = _mlp_out(hidden2, nw2, nb2)
    return (output, sel_idx)


# speedup vs baseline: 1.4941x; 1.0640x over previous
import functools

import jax
import jax.numpy as jnp
from jax import lax
from jax.experimental import pallas as pl

B = 8
K = 32768
H = 128
M = 1024
CHUNK = 8192
NCH = K // CHUNK


def _mlp_in_kernel(hr_ref, hi_ref, mask_ref, w_ref, b_ref, out_ref):
    g = pl.program_id(0)
    x = jnp.where(g < NCH, hr_ref[...], hi_ref[...])
    if mask_ref is not None:
        x = x * mask_ref[...]
    acc = jnp.dot(x, w_ref[...], preferred_element_type=jnp.float32)

    @pl.when(g == 0)
    def _():
        out_ref[...] = b_ref[...] + acc

    @pl.when(g > 0)
    def _():
        out_ref[...] += acc


def _mlp_in(hr, hi, mask, w, b):
    use_mask = mask is not None
    kern = functools.partial(_mlp_in_kernel) if use_mask else (
        lambda hr_ref, hi_ref, w_ref, b_ref, out_ref: _mlp_in_kernel(
            hr_ref, hi_ref, None, w_ref, b_ref, out_ref))
    in_specs = [
        pl.BlockSpec((B, CHUNK), lambda g: (0, jnp.where(g < NCH, g, 0))),
        pl.BlockSpec((B, CHUNK), lambda g: (0, jnp.where(g < NCH, 0, g - NCH))),
    ]
    args = [hr, hi]
    if use_mask:
        in_specs.append(pl.BlockSpec((B, CHUNK), lambda g: (0, lax.rem(g, NCH))))
        args.append(mask)
    in_specs += [
        pl.BlockSpec((CHUNK, H), lambda g: (g, 0)),
        pl.BlockSpec((1, H), lambda g: (0, 0)),
    ]
    args += [w, b.reshape(1, H)]
    out = pl.pallas_call(
        kern,
        grid=(2 * NCH,),
        in_specs=in_specs,
        out_specs=pl.BlockSpec((B, H), lambda g: (0, 0)),
        out_shape=jax.ShapeDtypeStruct((B, H), jnp.float32),
    )(*args)
    return jnp.maximum(out, 0.0)


def _mlp_out_kernel(h_ref, w_ref, b_ref, out_ref):
    out_ref[...] = b_ref[...] + jnp.dot(
        h_ref[...], w_ref[...], preferred_element_type=jnp.float32)


def _mlp_out(h, w, b):
    return pl.pallas_call(
        _mlp_out_kernel,
        grid=(NCH,),
        in_specs=[
            pl.BlockSpec((B, H), lambda g: (0, 0)),
            pl.BlockSpec((H, CHUNK), lambda g: (0, g)),
            pl.BlockSpec((1, CHUNK), lambda g: (0, g)),
        ],
        out_specs=pl.BlockSpec((B, CHUNK), lambda g: (0, g)),
        out_shape=jax.ShapeDtypeStruct((B, K), jnp.float32),
    )(h, w, b.reshape(1, K))


def _ce_pass(v, idx, s, dirsize):
    _, w = v.shape
    pos = lax.broadcasted_iota(jnp.int32, (1, w), 1)
    low = (pos & s) == 0
    desc = (pos & dirsize) == 0
    pv = jnp.where(low, jnp.roll(v, -s, axis=1), jnp.roll(v, s, axis=1))
    pi = jnp.where(low, jnp.roll(idx, -s, axis=1), jnp.roll(idx, s, axis=1))
    before = (v > pv) | ((v == pv) & (idx < pi))
    keep = before == (low == desc)
    return jnp.where(keep, v, pv), jnp.where(keep, idx, pi)


def _winner_pass(v, idx, half):
    _, w = v.shape
    pos = lax.broadcasted_iota(jnp.int32, (1, w), 1)
    low = (pos & half) == 0
    pv = jnp.where(low, jnp.roll(v, -half, axis=1), jnp.roll(v, half, axis=1))
    pi = jnp.where(low, jnp.roll(idx, -half, axis=1), jnp.roll(idx, half, axis=1))
    before = (v > pv) | ((v == pv) & (idx < pi))
    keep = before == low
    return jnp.where(keep, v, pv), jnp.where(keep, idx, pi)


def _drop_high_halves(x, half):
    b, w = x.shape
    nb = w // (2 * half)
    return x.reshape(b * nb, 2, half)[:, 0, :].reshape(b, w // 2)


def _topk_kernel(scores_ref, idx_ref, mask_ref, vals_ref):
    v = scores_ref[...]
    idx = lax.broadcasted_iota(jnp.int32, (B, K), 1)

    for k in range(1, 11):
        for j in reversed(range(k)):
            v, idx = _ce_pass(v, idx, 1 << j, 1 << k)

    w = K
    while w > M:
        v, idx = _winner_pass(v, idx, M)
        v = _drop_high_halves(v, M)
        idx = _drop_high_halves(idx, M)
        w //= 2
        for j in reversed(range(10)):
            v, idx = _ce_pass(v, idx, 1 << j, M)

    idx_ref[...] = idx
    vals_ref[...] = v

    sv = scores_ref[...]
    t = v[:, M - 1:M]
    ti = idx[:, M - 1:M]
    pos = lax.broadcasted_iota(jnp.int32, (B, K), 1)
    sel = (sv > t) | ((sv == t) & (pos <= ti))
    mask_ref[...] = sel.astype(jnp.float32)


def _topk(scores):
    return pl.pallas_call(
        _topk_kernel,
        in_specs=[pl.BlockSpec((B, K), lambda: (0, 0))],
        out_specs=[
            pl.BlockSpec((B, M), lambda: (0, 0)),
            pl.BlockSpec((B, K), lambda: (0, 0)),
            pl.BlockSpec((B, M), lambda: (0, 0)),
        ],
        out_shape=[
            jax.ShapeDtypeStruct((B, M), jnp.int32),
            jax.ShapeDtypeStruct((B, K), jnp.float32),
            jax.ShapeDtypeStruct((B, M), jnp.float32),
        ],
    )(scores)


def kernel(h_real, h_imag, iw1, ib1, iw2, ib2, nw1, nb1, nw2, nb2):
    hidden = _mlp_in(h_real, h_imag, None, iw1, ib1)
    scores = _mlp_out(hidden, iw2, ib2)
    sel_idx, mask, _vals = _topk(scores)
    hidden2 = _mlp_in(h_real, h_imag, mask, nw1, nb1)
    output = _mlp_out(hidden2, nw2, nb2)
    return (output, sel_idx)


# speedup vs baseline: 1.5122x; 1.0121x over previous
import functools

import jax
import jax.numpy as jnp
from jax import lax
from jax.experimental import pallas as pl

B = 8
K = 32768
H = 128
M = 1024
CHUNK = 16384
NCH = K // CHUNK


def _mlp_in_kernel(hr_ref, hi_ref, mask_ref, w_ref, b_ref, out_ref):
    g = pl.program_id(0)
    x = jnp.where(g < NCH, hr_ref[...], hi_ref[...])
    if mask_ref is not None:
        x = x * mask_ref[...]
    acc = jnp.dot(x, w_ref[...], preferred_element_type=jnp.float32)

    @pl.when(g == 0)
    def _():
        out_ref[...] = b_ref[...] + acc

    @pl.when(g > 0)
    def _():
        out_ref[...] += acc


def _mlp_in(hr, hi, mask, w, b):
    use_mask = mask is not None
    kern = functools.partial(_mlp_in_kernel) if use_mask else (
        lambda hr_ref, hi_ref, w_ref, b_ref, out_ref: _mlp_in_kernel(
            hr_ref, hi_ref, None, w_ref, b_ref, out_ref))
    in_specs = [
        pl.BlockSpec((B, CHUNK), lambda g: (0, jnp.where(g < NCH, g, 0))),
        pl.BlockSpec((B, CHUNK), lambda g: (0, jnp.where(g < NCH, 0, g - NCH))),
    ]
    args = [hr, hi]
    if use_mask:
        in_specs.append(pl.BlockSpec((B, CHUNK), lambda g: (0, lax.rem(g, NCH))))
        args.append(mask)
    in_specs += [
        pl.BlockSpec((CHUNK, H), lambda g: (g, 0)),
        pl.BlockSpec((1, H), lambda g: (0, 0)),
    ]
    args += [w, b.reshape(1, H)]
    out = pl.pallas_call(
        kern,
        grid=(2 * NCH,),
        in_specs=in_specs,
        out_specs=pl.BlockSpec((B, H), lambda g: (0, 0)),
        out_shape=jax.ShapeDtypeStruct((B, H), jnp.float32),
    )(*args)
    return jnp.maximum(out, 0.0)


def _mlp_out_kernel(h_ref, w_ref, b_ref, out_ref):
    out_ref[...] = b_ref[...] + jnp.dot(
        h_ref[...], w_ref[...], preferred_element_type=jnp.float32)


def _mlp_out(h, w, b):
    return pl.pallas_call(
        _mlp_out_kernel,
        grid=(NCH,),
        in_specs=[
            pl.BlockSpec((B, H), lambda g: (0, 0)),
            pl.BlockSpec((H, CHUNK), lambda g: (0, g)),
            pl.BlockSpec((1, CHUNK), lambda g: (0, g)),
        ],
        out_specs=pl.BlockSpec((B, CHUNK), lambda g: (0, g)),
        out_shape=jax.ShapeDtypeStruct((B, K), jnp.float32),
    )(h, w, b.reshape(1, K))


def _ce_pass(v, idx, s, dirsize):
    _, w = v.shape
    pos = lax.broadcasted_iota(jnp.int32, (1, w), 1)
    low = (pos & s) == 0
    desc = (pos & dirsize) == 0
    pv = jnp.where(low, jnp.roll(v, -s, axis=1), jnp.roll(v, s, axis=1))
    pi = jnp.where(low, jnp.roll(idx, -s, axis=1), jnp.roll(idx, s, axis=1))
    before = (v > pv) | ((v == pv) & (idx < pi))
    keep = before == (low == desc)
    return jnp.where(keep, v, pv), jnp.where(keep, idx, pi)


def _winner_pass(v, idx, half):
    _, w = v.shape
    pos = lax.broadcasted_iota(jnp.int32, (1, w), 1)
    low = (pos & half) == 0
    pv = jnp.where(low, jnp.roll(v, -half, axis=1), jnp.roll(v, half, axis=1))
    pi = jnp.where(low, jnp.roll(idx, -half, axis=1), jnp.roll(idx, half, axis=1))
    before = (v > pv) | ((v == pv) & (idx < pi))
    keep = before == low
    return jnp.where(keep, v, pv), jnp.where(keep, idx, pi)


def _drop_high_halves(x, half):
    b, w = x.shape
    nb = w // (2 * half)
    return x.reshape(b * nb, 2, half)[:, 0, :].reshape(b, w // 2)


def _topk_kernel(scores_ref, idx_ref, mask_ref, vals_ref):
    v = scores_ref[...]
    idx = lax.broadcasted_iota(jnp.int32, (B, K), 1)

    for k in range(1, 11):
        for j in reversed(range(k)):
            v, idx = _ce_pass(v, idx, 1 << j, 1 << k)

    w = K
    while w > M:
        v, idx = _winner_pass(v, idx, M)
        v = _drop_high_halves(v, M)
        idx = _drop_high_halves(idx, M)
        w //= 2
        for j in reversed(range(10)):
            v, idx = _ce_pass(v, idx, 1 << j, M)

    idx_ref[...] = idx
    vals_ref[...] = v

    sv = scores_ref[...]
    t = v[:, M - 1:M]
    ti = idx[:, M - 1:M]
    pos = lax.broadcasted_iota(jnp.int32, (B, K), 1)
    sel = (sv > t) | ((sv == t) & (pos <= ti))
    mask_ref[...] = sel.astype(jnp.float32)


def _topk(scores):
    return pl.pallas_call(
        _topk_kernel,
        in_specs=[pl.BlockSpec((B, K), lambda: (0, 0))],
        out_specs=[
            pl.BlockSpec((B, M), lambda: (0, 0)),
            pl.BlockSpec((B, K), lambda: (0, 0)),
            pl.BlockSpec((B, M), lambda: (0, 0)),
        ],
        out_shape=[
            jax.ShapeDtypeStruct((B, M), jnp.int32),
            jax.ShapeDtypeStruct((B, K), jnp.float32),
            jax.ShapeDtypeStruct((B, M), jnp.float32),
        ],
    )(scores)


def kernel(h_real, h_imag, iw1, ib1, iw2, ib2, nw1, nb1, nw2, nb2):
    hidden = _mlp_in(h_real, h_imag, None, iw1, ib1)
    scores = _mlp_out(hidden, iw2, ib2)
    sel_idx, mask, _vals = _topk(scores)
    hidden2 = _mlp_in(h_real, h_imag, mask, nw1, nb1)
    output = _mlp_out(hidden2, nw2, nb2)
    return (output, sel_idx)
